# baseline (speedup 1.0000x reference)
import functools

import jax
import jax.numpy as jnp
from jax import lax
from jax.experimental import pallas as pl
from jax.experimental.pallas import tpu as pltpu

_NORMALS = (16.0, 32.0, 64.0, 128.0, 256.0)
_SIZES = (64, 32, 16, 8, 4)
_PRE_NMS_TOP_N = 1000
_NUM_CLASSES = 80


def _level_body(scls_ref, aux_ref, loc_ref, out_ref, kref, *, n, kpad, normal):
    s = scls_ref[0]
    m = jnp.max(s, axis=0, keepdims=True)
    cidx = lax.broadcasted_iota(jnp.int32, (_NUM_CLASSES, n), 0)
    am = jnp.min(jnp.where(s == m, cidx, 127), axis=0, keepdims=True)
    ctr = aux_ref[0, 5:6, :]
    cand = (m * ctr) > 0.2
    idx_row = lax.broadcasted_iota(jnp.int32, (1, n), 1)
    kf = lax.bitcast_convert_type(m, jnp.int32)
    keys = jnp.where(cand, kf, -1 - idx_row)

    locx = loc_ref[0, 0:1, :]
    locy = loc_ref[0, 1:2, :]
    regs = aux_ref[0, 0:5, :]
    reg3 = (regs * (regs * regs)) * normal
    dt = jnp.concatenate([
        locx,
        locy,
        locx - reg3[0:1],
        locy - reg3[1:2],
        locx - reg3[2:3],
        locy - reg3[3:4],
        reg3[4:5],
        m,
        (am + 1).astype(jnp.float32),
        cand.astype(jnp.float32),
        jnp.zeros((6, n), jnp.float32),
    ], axis=0)

    bw = 128 if n >= 128 else n
    nb = n // bw
    kcd = keys.reshape(nb, bw)
    kref[...] = kcd
    kt = kcd.T
    riota = lax.broadcasted_iota(jnp.int32, (1, kpad), 1)
    jl = lax.broadcasted_iota(jnp.int32, (bw, bw), 1)
    il = lax.broadcasted_iota(jnp.int32, (bw, bw), 0)
    out_acc = jnp.zeros((16, kpad), jnp.float32)
    for ib in range(nb):
        ki = kt[:, ib:ib + 1]

        def body_ge(jc, acc, ki=ki):
            kj = kref[pl.ds(jc, 1), :]
            return acc + (kj >= ki).astype(jnp.int32)

        def body_gt(jc, acc, ki=ki):
            kj = kref[pl.ds(jc, 1), :]
            return acc + (kj > ki).astype(jnp.int32)

        acc = jnp.zeros((bw, bw), jnp.int32)
        acc = lax.fori_loop(0, ib, body_ge, acc)
        acc = lax.fori_loop(ib + 1, nb, body_gt, acc)
        kjd = kcd[ib:ib + 1, :]
        cd = (kjd > ki) | ((kjd == ki) & (jl < il))
        acc = acc + cd.astype(jnp.int32)
        rank = jnp.sum(acc, axis=1, keepdims=True)
        oh = (rank == riota).astype(jnp.float32)
        dtb = dt[:, ib * bw:(ib + 1) * bw]
        out_acc = out_acc + lax.dot(
            dtb, oh, precision=lax.Precision.HIGHEST,
            preferred_element_type=jnp.float32)
    out_ref[0] = out_acc


def _run_level(scls, aux, loc, *, n, kpad, normal):
    bw = 128 if n >= 128 else n
    return pl.pallas_call(
        functools.partial(_level_body, n=n, kpad=kpad, normal=normal),
        grid=(2,),
        scratch_shapes=[pltpu.VMEM((n // bw, bw), jnp.int32)],
        in_specs=[
            pl.BlockSpec((1, _NUM_CLASSES, n), lambda b: (b, 0, 0)),
            pl.BlockSpec((1, 8, n), lambda b: (b, 0, 0)),
            pl.BlockSpec((1, 8, n), lambda b: (0, 0, 0)),
        ],
        out_specs=pl.BlockSpec((1, 16, kpad), lambda b: (b, 0, 0)),
        out_shape=jax.ShapeDtypeStruct((2, 16, kpad), jnp.float32),
    )(scls, aux, loc)


def kernel(locations_0, locations_1, locations_2, locations_3, locations_4,
           box_cls_0, box_cls_1, box_cls_2, box_cls_3, box_cls_4,
           box_regression_0, box_regression_1, box_regression_2,
           box_regression_3, box_regression_4,
           centerness_0, centerness_1, centerness_2, centerness_3,
           centerness_4, image_sizes):
    locs = [locations_0, locations_1, locations_2, locations_3, locations_4]
    clss = [box_cls_0, box_cls_1, box_cls_2, box_cls_3, box_cls_4]
    regs = [box_regression_0, box_regression_1, box_regression_2,
            box_regression_3, box_regression_4]
    ctrs = [centerness_0, centerness_1, centerness_2, centerness_3,
            centerness_4]
    outs, labels, levels, valids = [], [], [], []
    for l, s in enumerate(_SIZES):
        n = s * s
        k = min(_PRE_NMS_TOP_N, n)
        kpad = max(k, ((k + 127) // 128) * 128) if n >= 128 else k
        scls = jax.nn.sigmoid(clss[l].reshape(2, _NUM_CLASSES, n))
        ctr_s = jax.nn.sigmoid(ctrs[l].reshape(2, 1, n))
        aux = jnp.concatenate(
            [regs[l].reshape(2, 5, n), ctr_s, jnp.zeros((2, 2, n), jnp.float32)],
            axis=1)
        loc3 = jnp.concatenate(
            [locs[l].T.reshape(1, 2, n), jnp.zeros((1, 6, n), jnp.float32)],
            axis=1)
        o16 = _run_level(scls, aux, loc3, n=n, kpad=kpad, normal=_NORMALS[l])
        o16 = o16[:, :, :k]
        outs.append(jnp.transpose(o16[:, 0:8, :], (0, 2, 1)))
        labels.append(jnp.round(o16[:, 8, :]).astype(jnp.int32))
        levels.append(jnp.full((2, k), l, jnp.int32))
        valids.append(o16[:, 9, :] > 0.5)
    out = jnp.concatenate(outs, axis=1)
    return (out, jnp.concatenate(labels, axis=1),
            jnp.concatenate(levels, axis=1), jnp.concatenate(valids, axis=1))


# speedup vs baseline: 3.0489x; 3.0489x over previous
import functools

import jax
import jax.numpy as jnp
from jax import lax
from jax.experimental import pallas as pl
from jax.experimental.pallas import tpu as pltpu

_NUM_CLASSES = 80
_SEGS = (
    (0, 4096, 4096, 1000, 1024, 0, 16.0),
    (4096, 1024, 1024, 1000, 1024, 1024, 32.0),
    (5120, 256, 256, 256, 256, 2048, 64.0),
    (5376, 64, 128, 64, 64, 2304, 128.0),
    (5504, 16, 16, 16, 16, 2368, 256.0),
)
_NTOT = 5520
_KTOT = 2384


def _rank_onehot(keys, kref, n):
    bw = 128 if n >= 128 else n
    nb = n // bw
    kcd = keys.reshape(nb, bw)
    if kref is not None:
        kref[...] = kcd
    kt = kcd.T
    jl = lax.broadcasted_iota(jnp.int32, (bw, bw), 1)
    il = lax.broadcasted_iota(jnp.int32, (bw, bw), 0)
    ranks = []
    for ib in range(nb):
        ki = kt[:, ib:ib + 1]
        acc = jnp.zeros((bw, bw), jnp.int16)
        if kref is None:
            for jc in range(ib):
                kj = kcd[jc:jc + 1, :]
                acc = acc + (kj >= ki).astype(jnp.int16)
            for jc in range(ib + 1, nb):
                kj = kcd[jc:jc + 1, :]
                acc = acc + (kj > ki).astype(jnp.int16)
        else:
            def body_ge(jc, a, ki=ki):
                kj = kref[pl.ds(jc, 1), :]
                return a + (kj >= ki).astype(jnp.int16)

            def body_gt(jc, a, ki=ki):
                kj = kref[pl.ds(jc, 1), :]
                return a + (kj > ki).astype(jnp.int16)

            acc = lax.fori_loop(0, ib, body_ge, acc, unroll=4)
            acc = lax.fori_loop(ib + 1, nb, body_gt, acc, unroll=4)
        kjd = kcd[ib:ib + 1, :]
        cd = (kjd > ki) | ((kjd == ki) & (jl < il))
        acc = acc + cd.astype(jnp.int16)
        rank = jnp.sum(acc.astype(jnp.int32), axis=1, keepdims=True)
        ranks.append(rank)
    return ranks, bw, nb


def _body(scls_ref, aux_ref, out_ref, kref0, kref1):
    krefs = {0: kref0, 1: kref1}
    for li, (off, n, _, _, kpad, roff, normal) in enumerate(_SEGS):
        s = scls_ref[0, :, off:off + n]
        m = jnp.max(s, axis=0, keepdims=True)
        cidx = lax.broadcasted_iota(jnp.int32, (_NUM_CLASSES, n), 0)
        am = jnp.min(jnp.where(s == m, cidx, 127), axis=0, keepdims=True)
        ctr = aux_ref[0, 5:6, off:off + n]
        cand = (m * ctr) > 0.2
        idx_row = lax.broadcasted_iota(jnp.int32, (1, n), 1)
        kf = lax.bitcast_convert_type(m, jnp.int32)
        keys = jnp.where(cand, kf, -1 - idx_row)

        locx = aux_ref[0, 6:7, off:off + n]
        locy = aux_ref[0, 7:8, off:off + n]
        regs = aux_ref[0, 0:5, off:off + n]
        reg3 = (regs * (regs * regs)) * normal
        dt = jnp.concatenate([
            locx,
            locy,
            locx - reg3[0:1],
            locy - reg3[1:2],
            locx - reg3[2:3],
            locy - reg3[3:4],
            reg3[4:5],
            m,
            (am + 1).astype(jnp.float32),
            cand.astype(jnp.float32),
            jnp.zeros((6, n), jnp.float32),
        ], axis=0)

        ranks, bw, nb = _rank_onehot(keys, krefs.get(li), n)
        riota = lax.broadcasted_iota(jnp.int32, (1, kpad), 1)
        out_acc = jnp.zeros((16, kpad), jnp.float32)
        for ib in range(nb):
            oh = (ranks[ib] == riota).astype(jnp.float32)
            dtb = dt[:, ib * bw:(ib + 1) * bw]
            out_acc = out_acc + lax.dot(
                dtb, oh, precision=lax.Precision.HIGHEST,
                preferred_element_type=jnp.float32)
        out_ref[0, roff:roff + kpad, :] = out_acc.T


def _postprocess(scls_all, aux_all):
    return pl.pallas_call(
        _body,
        grid=(2,),
        scratch_shapes=[pltpu.VMEM((32, 128), jnp.int32),
                        pltpu.VMEM((8, 128), jnp.int32)],
        in_specs=[
            pl.BlockSpec((1, _NUM_CLASSES, _NTOT), lambda b: (b, 0, 0)),
            pl.BlockSpec((1, 8, _NTOT), lambda b: (b, 0, 0)),
        ],
        out_specs=pl.BlockSpec((1, _KTOT, 16), lambda b: (b, 0, 0)),
        out_shape=jax.ShapeDtypeStruct((2, _KTOT, 16), jnp.float32),
    )(scls_all, aux_all)


def kernel(locations_0, locations_1, locations_2, locations_3, locations_4,
           box_cls_0, box_cls_1, box_cls_2, box_cls_3, box_cls_4,
           box_regression_0, box_regression_1, box_regression_2,
           box_regression_3, box_regression_4,
           centerness_0, centerness_1, centerness_2, centerness_3,
           centerness_4, image_sizes):
    locs = [locations_0, locations_1, locations_2, locations_3, locations_4]
    clss = [box_cls_0, box_cls_1, box_cls_2, box_cls_3, box_cls_4]
    regs = [box_regression_0, box_regression_1, box_regression_2,
            box_regression_3, box_regression_4]
    ctrs = [centerness_0, centerness_1, centerness_2, centerness_3,
            centerness_4]
    ns = [s[1] for s in _SEGS]
    pads = [s[2] - s[1] for s in _SEGS]

    def cat(parts, pad_shape_fn):
        chunks = []
        for p, pad in zip(parts, pads):
            chunks.append(p)
            if pad:
                chunks.append(jnp.zeros(pad_shape_fn(pad), jnp.float32))
        return jnp.concatenate(chunks, axis=-1)

    scls_all = jax.nn.sigmoid(cat(
        [c.reshape(2, _NUM_CLASSES, n) for c, n in zip(clss, ns)],
        lambda p: (2, _NUM_CLASSES, p)))
    ctr_all = jax.nn.sigmoid(cat(
        [c.reshape(2, 1, n) for c, n in zip(ctrs, ns)],
        lambda p: (2, 1, p)))
    regs_all = cat([r.reshape(2, 5, n) for r, n in zip(regs, ns)],
                   lambda p: (2, 5, p))
    loc_all = cat([l.T.reshape(1, 2, n) for l, n in zip(locs, ns)],
                  lambda p: (1, 2, p))
    aux_all = jnp.concatenate(
        [regs_all, ctr_all, jnp.broadcast_to(loc_all, (2, 2, _NTOT))], axis=1)

    o = _postprocess(scls_all, aux_all)
    parts = [o[:, roff:roff + k, :] for (_, _, _, k, _, roff, _) in _SEGS]
    big = jnp.concatenate(parts, axis=1)
    out = big[:, :, 0:8]
    labels = jnp.round(big[:, :, 8]).astype(jnp.int32)
    valids = big[:, :, 9] > 0.5
    lvl = jnp.concatenate(
        [jnp.full((2, s[3]), i, jnp.int32) for i, s in enumerate(_SEGS)],
        axis=1)
    return out, labels, lvl, valids


# speedup vs baseline: 3.6163x; 1.1861x over previous
import functools

import jax
import jax.numpy as jnp
from jax import lax
from jax.experimental import pallas as pl
from jax.experimental.pallas import tpu as pltpu

_NUM_CLASSES = 80
_SEGS = (
    (0, 4096, 4096, 1000, 1024, 0, 16.0),
    (4096, 1024, 1024, 1000, 1024, 1024, 32.0),
    (5120, 256, 256, 256, 256, 2048, 64.0),
    (5376, 64, 128, 64, 64, 2304, 128.0),
    (5504, 16, 16, 16, 16, 2368, 256.0),
)
_NTOT = 5520
_KTOT = 2384


def _rank_onehot(keys, kref, n):
    bw = 128 if n >= 128 else n
    nb = n // bw
    kcd = keys.reshape(nb, bw)
    if kref is not None:
        kref[...] = kcd
    kt = kcd.T
    jl = lax.broadcasted_iota(jnp.int32, (bw, bw), 1)
    il = lax.broadcasted_iota(jnp.int32, (bw, bw), 0)
    ranks = []
    for ib in range(nb):
        ki = kt[:, ib:ib + 1]
        acc = jnp.zeros((bw, bw), jnp.int16)
        if kref is None:
            for jc in range(ib):
                kj = kcd[jc:jc + 1, :]
                acc = acc + (kj >= ki).astype(jnp.int16)
            for jc in range(ib + 1, nb):
                kj = kcd[jc:jc + 1, :]
                acc = acc + (kj > ki).astype(jnp.int16)
        else:
            def body_ge(jc, a, ki=ki):
                kj = kref[pl.ds(jc, 1), :]
                return a + (kj >= ki).astype(jnp.int16)

            def body_gt(jc, a, ki=ki):
                kj = kref[pl.ds(jc, 1), :]
                return a + (kj > ki).astype(jnp.int16)

            acc = lax.fori_loop(0, ib, body_ge, acc, unroll=4)
            acc = lax.fori_loop(ib + 1, nb, body_gt, acc, unroll=4)
        kjd = kcd[ib:ib + 1, :]
        cd = (kjd > ki) | ((kjd == ki) & (jl < il))
        acc = acc + cd.astype(jnp.int16)
        rank = jnp.sum(acc.astype(jnp.int32), axis=1, keepdims=True)
        ranks.append(rank)
    return ranks, bw, nb


def _body(scls_ref, aux_ref, out_ref, kref0, kref1):
    krefs = {0: kref0, 1: kref1}
    for li, (off, n, _, _, kpad, roff, normal) in enumerate(_SEGS):
        s = scls_ref[0, :, off:off + n]
        m = jnp.max(s, axis=0, keepdims=True)
        cidx = lax.broadcasted_iota(jnp.int32, (_NUM_CLASSES, n), 0)
        am = jnp.min(jnp.where(s == m, cidx, 127), axis=0, keepdims=True)
        ctr = aux_ref[0, 5:6, off:off + n]
        cand = (m * ctr) > 0.2
        idx_row = lax.broadcasted_iota(jnp.int32, (1, n), 1)
        kf = lax.bitcast_convert_type(m, jnp.int32)
        keys = jnp.where(cand, kf, -1 - idx_row)

        locx = aux_ref[0, 6:7, off:off + n]
        locy = aux_ref[0, 7:8, off:off + n]
        regs = aux_ref[0, 0:5, off:off + n]
        reg3 = (regs * (regs * regs)) * normal
        dt = jnp.concatenate([
            locx,
            locy,
            locx - reg3[0:1],
            locy - reg3[1:2],
            locx - reg3[2:3],
            locy - reg3[3:4],
            reg3[4:5],
            m,
            (am + 1).astype(jnp.float32),
            cand.astype(jnp.float32),
            jnp.zeros((6, n), jnp.float32),
        ], axis=0)

        ranks, bw, nb = _rank_onehot(keys, krefs.get(li), n)
        dt_hi = dt.astype(jnp.bfloat16)
        dt_lo = (dt - dt_hi.astype(jnp.float32)).astype(jnp.bfloat16)
        riota = lax.broadcasted_iota(jnp.int32, (1, kpad), 1)
        out_acc = jnp.zeros((16, kpad), jnp.float32)
        for ib in range(nb):
            oh = (ranks[ib] == riota).astype(jnp.bfloat16)
            sl = slice(ib * bw, (ib + 1) * bw)
            out_acc = (out_acc
                       + lax.dot(dt_hi[:, sl], oh,
                                 preferred_element_type=jnp.float32)
                       + lax.dot(dt_lo[:, sl], oh,
                                 preferred_element_type=jnp.float32))
        out_ref[0, roff:roff + kpad, :] = out_acc.T


def _postprocess(scls_all, aux_all):
    return pl.pallas_call(
        _body,
        grid=(2,),
        scratch_shapes=[pltpu.VMEM((32, 128), jnp.int32),
                        pltpu.VMEM((8, 128), jnp.int32)],
        compiler_params=pltpu.CompilerParams(
            dimension_semantics=("parallel",)),
        in_specs=[
            pl.BlockSpec((1, _NUM_CLASSES, _NTOT), lambda b: (b, 0, 0)),
            pl.BlockSpec((1, 8, _NTOT), lambda b: (b, 0, 0)),
        ],
        out_specs=pl.BlockSpec((1, _KTOT, 16), lambda b: (b, 0, 0)),
        out_shape=jax.ShapeDtypeStruct((2, _KTOT, 16), jnp.float32),
    )(scls_all, aux_all)


def kernel(locations_0, locations_1, locations_2, locations_3, locations_4,
           box_cls_0, box_cls_1, box_cls_2, box_cls_3, box_cls_4,
           box_regression_0, box_regression_1, box_regression_2,
           box_regression_3, box_regression_4,
           centerness_0, centerness_1, centerness_2, centerness_3,
           centerness_4, image_sizes):
    locs = [locations_0, locations_1, locations_2, locations_3, locations_4]
    clss = [box_cls_0, box_cls_1, box_cls_2, box_cls_3, box_cls_4]
    regs = [box_regression_0, box_regression_1, box_regression_2,
            box_regression_3, box_regression_4]
    ctrs = [centerness_0, centerness_1, centerness_2, centerness_3,
            centerness_4]
    ns = [s[1] for s in _SEGS]
    pads = [s[2] - s[1] for s in _SEGS]

    def cat(parts, pad_shape_fn):
        chunks = []
        for p, pad in zip(parts, pads):
            chunks.append(p)
            if pad:
                chunks.append(jnp.zeros(pad_shape_fn(pad), jnp.float32))
        return jnp.concatenate(chunks, axis=-1)

    scls_all = jax.nn.sigmoid(cat(
        [c.reshape(2, _NUM_CLASSES, n) for c, n in zip(clss, ns)],
        lambda p: (2, _NUM_CLASSES, p)))
    ctr_all = jax.nn.sigmoid(cat(
        [c.reshape(2, 1, n) for c, n in zip(ctrs, ns)],
        lambda p: (2, 1, p)))
    regs_all = cat([r.reshape(2, 5, n) for r, n in zip(regs, ns)],
                   lambda p: (2, 5, p))
    loc_all = cat([l.T.reshape(1, 2, n) for l, n in zip(locs, ns)],
                  lambda p: (1, 2, p))
    aux_all = jnp.concatenate(
        [regs_all, ctr_all, jnp.broadcast_to(loc_all, (2, 2, _NTOT))], axis=1)

    o = _postprocess(scls_all, aux_all)
    parts = [o[:, roff:roff + k, :] for (_, _, _, k, _, roff, _) in _SEGS]
    big = jnp.concatenate(parts, axis=1)
    out = big[:, :, 0:8]
    labels = jnp.round(big[:, :, 8]).astype(jnp.int32)
    valids = big[:, :, 9] > 0.5
    lvl = jnp.concatenate(
        [jnp.full((2, s[3]), i, jnp.int32) for i, s in enumerate(_SEGS)],
        axis=1)
    return out, labels, lvl, valids
